# Initial kernel scaffold; baseline (speedup 1.0000x reference)
import jax
import jax.numpy as jnp
from jax.experimental import pallas as pl


def kernel(input, gt_box, im_info, W1, b1, Wc, bc, Wr, br):
    raise NotImplementedError("write your pallas kernel here")


# speedup vs baseline: 83.4780x; 83.4780x over previous
import functools

import numpy as np
import jax
import jax.numpy as jnp
from jax.experimental import pallas as pl
from jax.experimental.pallas import tpu as pltpu

_B = 2
_H = _W = 64
_A = 9
_N = _H * _W * _A
_ROWS = _N // 128
_PRE_N = 6000
_POST_N = 300
_NMS_T = 0.7
_MINSZ = 16.0
_STRIDE = 16


def _anchor_components():
    base = np.array([0.0, 0.0, _STRIDE - 1.0, _STRIDE - 1.0])
    w = base[2] - base[0] + 1.0
    h = base[3] - base[1] + 1.0
    xctr = base[0] + 0.5 * (w - 1.0)
    yctr = base[1] + 0.5 * (h - 1.0)
    size = w * h
    out = []
    for r in [0.5, 1.0, 2.0]:
        ws = np.round(np.sqrt(size / r))
        hs = np.round(ws * r)
        for s in [8, 16, 32]:
            w2 = ws * s
            h2 = hs * s
            out.append([xctr - 0.5 * (w2 - 1.0), yctr - 0.5 * (h2 - 1.0),
                        xctr + 0.5 * (w2 - 1.0), yctr + 0.5 * (h2 - 1.0)])
    basea = np.array(out, dtype=np.float32)
    sx = np.arange(_W, dtype=np.float32) * _STRIDE
    sy = np.arange(_H, dtype=np.float32) * _STRIDE
    SX, SY = np.meshgrid(sx, sy)
    shifts = np.stack([SX.ravel(), SY.ravel(), SX.ravel(), SY.ravel()], axis=1)
    anc = (shifts[:, None, :].astype(np.float32)
           + basea[None, :, :]).reshape(-1, 4).astype(np.float32)
    wa = anc[:, 2] - anc[:, 0] + np.float32(1.0)
    ha = anc[:, 3] - anc[:, 1] + np.float32(1.0)
    xa = anc[:, 0] + np.float32(0.5) * wa
    ya = anc[:, 1] + np.float32(0.5) * ha
    rs = lambda v: v.reshape(_ROWS, 128)
    return rs(wa), rs(ha), rs(xa), rs(ya)


_WA, _HA, _XA, _YA = _anchor_components()


def _rpn_core(p_ref, l0_ref, l1_ref, dx_ref, dy_ref, dw_ref, dh_ref,
              wa_ref, ha_ref, xa_ref, ya_ref, out_ref,
              keys_ref, x1_ref, y1_ref, x2_ref, y2_ref, ar_ref,
              el_ref, sup_ref):
    imw1 = p_ref[0, 0, 0]
    imh1 = p_ref[0, 0, 1]
    minsz = p_ref[0, 0, 2]
    bi = p_ref[0, 0, 3]

    l0 = l0_ref[0]
    l1 = l1_ref[0]
    m = jnp.maximum(l0, l1)
    e0 = jnp.exp(l0 - m)
    e1 = jnp.exp(l1 - m)
    fg = e1 / (e0 + e1)

    wa = wa_ref[...]
    ha = ha_ref[...]
    xa = xa_ref[...]
    ya = ya_ref[...]
    dxv = dx_ref[0]
    dyv = dy_ref[0]
    dwv = jnp.clip(dw_ref[0], -10.0, 4.135)
    dhv = jnp.clip(dh_ref[0], -10.0, 4.135)
    xc = dxv * wa + xa
    yc = dyv * ha + ya
    wp = jnp.exp(dwv) * wa
    hp = jnp.exp(dhv) * ha
    x1 = jnp.clip(xc - 0.5 * wp, 0.0, imw1)
    y1 = jnp.clip(yc - 0.5 * hp, 0.0, imh1)
    x2 = jnp.clip(xc + 0.5 * wp, 0.0, imw1)
    y2 = jnp.clip(yc + 0.5 * hp, 0.0, imh1)
    ww = x2 - x1 + 1.0
    hh = y2 - y1 + 1.0
    ok = jnp.logical_and(ww >= minsz, hh >= minsz)
    sc = jnp.where(ok, fg, jnp.float32(-1e9))

    sb = jax.lax.bitcast_convert_type(sc, jnp.int32)
    keys = jnp.where(sb >= 0, sb, sb ^ jnp.int32(0x7FFFFFFF))
    keys_ref[...] = keys
    x1_ref[...] = x1
    y1_ref[...] = y1
    x2_ref[...] = x2
    y2_ref[...] = y2
    ar_ref[...] = ww * hh

    lin = (jax.lax.broadcasted_iota(jnp.int32, (_ROWS, 128), 0) * 128
           + jax.lax.broadcasted_iota(jnp.int32, (_ROWS, 128), 1))
    minint = jnp.int32(-2147483648)

    def tbody(i, tu):
        tu2 = tu | (jnp.int32(1) << (31 - i))
        cnt = jnp.sum((keys >= (tu2 ^ minint)).astype(jnp.int32))
        return jnp.where(cnt >= _PRE_N, tu2, tu)

    tu = jax.lax.fori_loop(0, 32, tbody, jnp.int32(0))
    tkey = tu ^ minint
    cnt_gt = jnp.sum((keys > tkey).astype(jnp.int32))
    rbudget = _PRE_N - cnt_gt
    tie = keys == tkey

    def jbody(i, ju):
        ju2 = ju | (jnp.int32(1) << (16 - i))
        cnt = jnp.sum(jnp.logical_and(tie, lin < ju2).astype(jnp.int32))
        return jnp.where(cnt < rbudget, ju2, ju)

    ju = jax.lax.fori_loop(0, 17, jbody, jnp.int32(0))
    elig = jnp.logical_or(
        keys > tkey,
        jnp.logical_and(rbudget > 0, jnp.logical_and(tie, lin <= ju)))
    el_ref[...] = elig.astype(jnp.int32)
    sup_ref[...] = jnp.zeros((_ROWS, 128), jnp.int32)

    lane = jax.lax.broadcasted_iota(jnp.int32, (_POST_N + 4, 128), 1)
    out_ref[0] = jnp.where(lane == 0, bi, 0.0)
    lane1 = jax.lax.broadcasted_iota(jnp.int32, (1, 128), 1)

    def nms_body(_, cnt):
        alive = jnp.logical_and(el_ref[...] == 1, sup_ref[...] == 0)
        kk = keys_ref[...]
        mk = jnp.max(jnp.where(alive, kk, minint))
        anyv = mk > minint
        eq = jnp.logical_and(alive, kk == mk)
        idxv = jnp.min(jnp.where(eq, lin, jnp.int32(0x40000000)))
        one = lin == idxv
        bx1 = jnp.sum(jnp.where(one, x1_ref[...], 0.0))
        by1 = jnp.sum(jnp.where(one, y1_ref[...], 0.0))
        bx2 = jnp.sum(jnp.where(one, x2_ref[...], 0.0))
        by2 = jnp.sum(jnp.where(one, y2_ref[...], 0.0))
        bar = jnp.sum(jnp.where(one, ar_ref[...], 0.0))

        @pl.when(anyv)
        def _():
            xx1 = jnp.maximum(bx1, x1_ref[...])
            yy1 = jnp.maximum(by1, y1_ref[...])
            xx2 = jnp.minimum(bx2, x2_ref[...])
            yy2 = jnp.minimum(by2, y2_ref[...])
            inter = (jnp.maximum(0.0, xx2 - xx1 + 1.0)
                     * jnp.maximum(0.0, yy2 - yy1 + 1.0))
            iou = inter / (bar + ar_ref[...] - inter)
            sup_ref[...] = jnp.where(iou > _NMS_T, jnp.int32(1), sup_ref[...])
            row = jnp.where(lane1 == 0, bi,
                  jnp.where(lane1 == 1, bx1,
                  jnp.where(lane1 == 2, by1,
                  jnp.where(lane1 == 3, bx2,
                  jnp.where(lane1 == 4, by2, 0.0)))))
            out_ref[0, pl.ds(cnt, 1), :] = row

        return cnt + jnp.where(anyv, 1, 0)

    jax.lax.fori_loop(0, _POST_N, nms_body, jnp.int32(0))


def _conv_nchw(x, w, b):
    y = jax.lax.conv_general_dilated(x, w, (1, 1), 'SAME',
                                     dimension_numbers=('NCHW', 'OIHW', 'NCHW'))
    return y + b[None, :, None, None]


@jax.jit
def kernel(input, gt_box, im_info, W1, b1, Wc, bc, Wr, br):
    x = jax.nn.relu(_conv_nchw(input, W1, b1))
    cls = jnp.transpose(_conv_nchw(x, Wc, bc), (0, 2, 3, 1)).reshape(_B, _H * _W, 2 * _A)
    reg = jnp.transpose(_conv_nchw(x, Wr, br), (0, 2, 3, 1)).reshape(_B, _H * _W, 4 * _A)

    rs = lambda v: v.reshape(_B, _ROWS, 128)
    l0 = rs(cls[:, :, 0::2])
    l1 = rs(cls[:, :, 1::2])
    dxv = rs(reg[:, :, 0::4])
    dyv = rs(reg[:, :, 1::4])
    dwv = rs(reg[:, :, 2::4])
    dhv = rs(reg[:, :, 3::4])

    params = jnp.stack([
        im_info[:, 1] - 1.0,
        im_info[:, 0] - 1.0,
        _MINSZ * im_info[:, 2],
        jnp.arange(_B, dtype=jnp.float32),
    ], axis=1).reshape(_B, 1, 4)

    grid_spec = pltpu.PrefetchScalarGridSpec(
        num_scalar_prefetch=0,
        grid=(_B,),
        in_specs=[
            pl.BlockSpec((1, 1, 4), lambda b: (b, 0, 0), memory_space=pltpu.SMEM),
            pl.BlockSpec((1, _ROWS, 128), lambda b: (b, 0, 0)),
            pl.BlockSpec((1, _ROWS, 128), lambda b: (b, 0, 0)),
            pl.BlockSpec((1, _ROWS, 128), lambda b: (b, 0, 0)),
            pl.BlockSpec((1, _ROWS, 128), lambda b: (b, 0, 0)),
            pl.BlockSpec((1, _ROWS, 128), lambda b: (b, 0, 0)),
            pl.BlockSpec((1, _ROWS, 128), lambda b: (b, 0, 0)),
            pl.BlockSpec((_ROWS, 128), lambda b: (0, 0)),
            pl.BlockSpec((_ROWS, 128), lambda b: (0, 0)),
            pl.BlockSpec((_ROWS, 128), lambda b: (0, 0)),
            pl.BlockSpec((_ROWS, 128), lambda b: (0, 0)),
        ],
        out_specs=pl.BlockSpec((1, _POST_N + 4, 128), lambda b: (b, 0, 0)),
        scratch_shapes=[
            pltpu.VMEM((_ROWS, 128), jnp.int32),
            pltpu.VMEM((_ROWS, 128), jnp.float32),
            pltpu.VMEM((_ROWS, 128), jnp.float32),
            pltpu.VMEM((_ROWS, 128), jnp.float32),
            pltpu.VMEM((_ROWS, 128), jnp.float32),
            pltpu.VMEM((_ROWS, 128), jnp.float32),
            pltpu.VMEM((_ROWS, 128), jnp.int32),
            pltpu.VMEM((_ROWS, 128), jnp.int32),
        ],
    )
    rois_wide = pl.pallas_call(
        _rpn_core,
        grid_spec=grid_spec,
        out_shape=jax.ShapeDtypeStruct((_B, _POST_N + 4, 128), jnp.float32),
    )(params, l0, l1, dxv, dyv, dwv, dhv,
      jnp.asarray(_WA), jnp.asarray(_HA), jnp.asarray(_XA), jnp.asarray(_YA))
    return rois_wide[:, :_POST_N, :5]
